# baseline (speedup 1.0000x reference)
import functools

import jax
import jax.numpy as jnp
from jax import lax
from jax.experimental import pallas as pl
from jax.experimental.pallas import tpu as pltpu
from jax.experimental.pallas import tpu_sc as plsc

_D = 2048
_B = 4 * 8192
_SCALE = float(_D) ** 0.5
_NC, _NS, _L = 2, 16, 16
_NW = _NC * _NS
_BPW = _B // _NW
_C = 8
_NCHUNK = _BPW // _C
_NBUF = 4

_mesh = plsc.VectorSubcoreMesh(core_axis_name="c", subcore_axis_name="s")


@functools.partial(
    pl.kernel,
    mesh=_mesh,
    out_type=jax.ShapeDtypeStruct((_B, _D), jnp.float32),
    scratch_types=(
        [pltpu.VMEM((_BPW,), jnp.int32)]
        + [pltpu.VMEM((_C, _D), jnp.float32)] * _NBUF
        + [pltpu.SemaphoreType.DMA] * (2 * _NBUF)
    ),
)
def _emb_lookup(ids_hbm, table_hbm, out_hbm, idx_v, *bufs_and_sems):
    bufs = bufs_and_sems[:_NBUF]
    gsems = bufs_and_sems[_NBUF:2 * _NBUF]
    wsems = bufs_and_sems[2 * _NBUF:]

    wid = lax.axis_index("s") * _NC + lax.axis_index("c")
    base = wid * _BPW
    pltpu.sync_copy(ids_hbm.at[pl.ds(base, _BPW)], idx_v)

    def gather(c, b):
        pltpu.async_copy(table_hbm.at[idx_v.at[pl.ds(c * _C, _C)]], bufs[b], gsems[b])

    def wait_gather(c, b):
        pltpu.make_async_copy(
            table_hbm.at[idx_v.at[pl.ds(c * _C, _C)]], bufs[b], gsems[b]
        ).wait()

    def write(c, b):
        pltpu.async_copy(bufs[b], out_hbm.at[pl.ds(base + c * _C, _C)], wsems[b])

    def wait_write(c, b):
        pltpu.make_async_copy(
            bufs[b], out_hbm.at[pl.ds(base + c * _C, _C)], wsems[b]
        ).wait()

    def scale(b):
        buf = bufs[b]

        @plsc.parallel_loop(0, _C, 1, unroll=1)
        def _(r):
            for j in range(_D // _L):
                sl = pl.ds(j * _L, _L)
                buf[r, sl] = buf[r, sl] * _SCALE

    for c in range(_NBUF - 1):
        gather(c, c)

    def step(g, b):
        @pl.when(g >= 1)
        def _():
            wait_write(g - 1, (b - 1) % _NBUF)

        @pl.when(g + _NBUF - 1 < _NCHUNK)
        def _():
            gather(g + _NBUF - 1, (b - 1) % _NBUF)

        wait_gather(g, b)
        scale(b)
        write(g, b)

    _MAIN = (_NCHUNK // _NBUF) * _NBUF

    def ring_body(i, carry):
        g0 = i * _NBUF
        for b in range(_NBUF):
            step(g0 + b, b)
        return carry

    lax.fori_loop(0, _MAIN // _NBUF, ring_body, 0, unroll=False)
    for g in range(_MAIN, _NCHUNK):
        step(g, g % _NBUF)
    wait_write(_NCHUNK - 1, (_NCHUNK - 1) % _NBUF)


def kernel(input_ids, table):
    ids = input_ids.reshape(-1).astype(jnp.int32)
    out = _emb_lookup(ids, table)
    return out.reshape(*input_ids.shape, _D)


# speedup vs baseline: 1.0096x; 1.0096x over previous
import functools

import jax
import jax.numpy as jnp
from jax import lax
from jax.experimental import pallas as pl
from jax.experimental.pallas import tpu as pltpu
from jax.experimental.pallas import tpu_sc as plsc

_D = 2048
_B = 4 * 8192
_SCALE = float(_D) ** 0.5
_NC, _NS, _L = 2, 16, 16
_NW = _NC * _NS
_BPW = _B // _NW
_C = 16
_NCHUNK = _BPW // _C
_NBUF = 3
_PREF = 1
_SLACK = _NBUF - 1 - _PREF

_mesh = plsc.VectorSubcoreMesh(core_axis_name="c", subcore_axis_name="s")


@functools.partial(
    pl.kernel,
    mesh=_mesh,
    out_type=jax.ShapeDtypeStruct((_B, _D), jnp.float32),
    scratch_types=(
        [pltpu.VMEM((_BPW,), jnp.int32)]
        + [pltpu.VMEM((_C, _D), jnp.float32)] * _NBUF
        + [pltpu.SemaphoreType.DMA] * (2 * _NBUF)
    ),
)
def _emb_lookup(ids_hbm, table_hbm, out_hbm, idx_v, *bufs_and_sems):
    bufs = bufs_and_sems[:_NBUF]
    gsems = bufs_and_sems[_NBUF:2 * _NBUF]
    wsems = bufs_and_sems[2 * _NBUF:]

    wid = lax.axis_index("s") * _NC + lax.axis_index("c")
    base = wid * _BPW
    pltpu.sync_copy(ids_hbm.at[pl.ds(base, _BPW)], idx_v)

    def gather(c, b):
        pltpu.async_copy(table_hbm.at[idx_v.at[pl.ds(c * _C, _C)]], bufs[b], gsems[b])

    def wait_gather(c, b):
        pltpu.make_async_copy(
            table_hbm.at[idx_v.at[pl.ds(c * _C, _C)]], bufs[b], gsems[b]
        ).wait()

    def write(c, b):
        pltpu.async_copy(bufs[b], out_hbm.at[pl.ds(base + c * _C, _C)], wsems[b])

    def wait_write(c, b):
        pltpu.make_async_copy(
            bufs[b], out_hbm.at[pl.ds(base + c * _C, _C)], wsems[b]
        ).wait()

    def scale(b):
        buf = bufs[b]

        @plsc.parallel_loop(0, _C, 1, unroll=1)
        def _(r):
            for j in range(_D // _L):
                sl = pl.ds(j * _L, _L)
                buf[r, sl] = buf[r, sl] * _SCALE

    for c in range(_PREF):
        gather(c, c)

    def step(g, b):
        @pl.when(g >= 1 + _SLACK)
        def _():
            wait_write(g - 1 - _SLACK, (b - 1 - _SLACK) % _NBUF)

        @pl.when(g + _PREF < _NCHUNK)
        def _():
            gather(g + _PREF, (b + _PREF) % _NBUF)

        wait_gather(g, b)
        scale(b)
        write(g, b)

    _MAIN = (_NCHUNK // _NBUF) * _NBUF

    def ring_body(i, carry):
        g0 = i * _NBUF
        for b in range(_NBUF):
            step(g0 + b, b)
        return carry

    lax.fori_loop(0, _MAIN // _NBUF, ring_body, 0, unroll=False)
    for g in range(_MAIN, _NCHUNK):
        step(g, g % _NBUF)
    for g in range(_NCHUNK - 1 - _SLACK, _NCHUNK):
        wait_write(g, g % _NBUF)


def kernel(input_ids, table):
    ids = input_ids.reshape(-1).astype(jnp.int32)
    out = _emb_lookup(ids, table)
    return out.reshape(*input_ids.shape, _D)


# speedup vs baseline: 1.0228x; 1.0130x over previous
import functools

import jax
import jax.numpy as jnp
from jax import lax
from jax.experimental import pallas as pl
from jax.experimental.pallas import tpu as pltpu
from jax.experimental.pallas import tpu_sc as plsc

_D = 2048
_B = 4 * 8192
_SCALE = float(_D) ** 0.5
_NC, _NS, _L = 2, 16, 16
_NW = _NC * _NS
_BPW = _B // _NW
_C = 16
_NCHUNK = _BPW // _C
_NBUF = 2
_PREF = 1
_SLACK = _NBUF - 1 - _PREF

_mesh = plsc.VectorSubcoreMesh(core_axis_name="c", subcore_axis_name="s")


@functools.partial(
    pl.kernel,
    mesh=_mesh,
    out_type=jax.ShapeDtypeStruct((_B, _D), jnp.float32),
    scratch_types=(
        [pltpu.VMEM((_BPW,), jnp.int32)]
        + [pltpu.VMEM((_C, _D), jnp.float32)] * _NBUF
        + [pltpu.SemaphoreType.DMA] * (2 * _NBUF)
    ),
)
def _emb_lookup(ids_hbm, table_hbm, out_hbm, idx_v, *bufs_and_sems):
    bufs = bufs_and_sems[:_NBUF]
    gsems = bufs_and_sems[_NBUF:2 * _NBUF]
    wsems = bufs_and_sems[2 * _NBUF:]

    wid = lax.axis_index("s") * _NC + lax.axis_index("c")
    base = wid * _BPW
    pltpu.sync_copy(ids_hbm.at[pl.ds(base, _BPW)], idx_v)

    def gather(c, b):
        pltpu.async_copy(table_hbm.at[idx_v.at[pl.ds(c * _C, _C)]], bufs[b], gsems[b])

    def wait_gather(c, b):
        pltpu.make_async_copy(
            table_hbm.at[idx_v.at[pl.ds(c * _C, _C)]], bufs[b], gsems[b]
        ).wait()

    def write(c, b):
        pltpu.async_copy(bufs[b], out_hbm.at[pl.ds(base + c * _C, _C)], wsems[b])

    def wait_write(c, b):
        pltpu.make_async_copy(
            bufs[b], out_hbm.at[pl.ds(base + c * _C, _C)], wsems[b]
        ).wait()

    def scale(b):
        buf = bufs[b]

        @plsc.parallel_loop(0, _C, 1, unroll=1)
        def _(r):
            for j in range(_D // _L):
                sl = pl.ds(j * _L, _L)
                buf[r, sl] = buf[r, sl] * _SCALE

    for c in range(_PREF):
        gather(c, c)

    def step(g, b):
        @pl.when(g >= 1 + _SLACK)
        def _():
            wait_write(g - 1 - _SLACK, (b - 1 - _SLACK) % _NBUF)

        @pl.when(g + _PREF < _NCHUNK)
        def _():
            gather(g + _PREF, (b + _PREF) % _NBUF)

        wait_gather(g, b)
        scale(b)
        write(g, b)

    _MAIN = (_NCHUNK // _NBUF) * _NBUF

    def ring_body(i, carry):
        g0 = i * _NBUF
        for b in range(_NBUF):
            step(g0 + b, b)
        return carry

    lax.fori_loop(0, _MAIN // _NBUF, ring_body, 0, unroll=False)
    for g in range(_MAIN, _NCHUNK):
        step(g, g % _NBUF)
    for g in range(_NCHUNK - 1 - _SLACK, _NCHUNK):
        wait_write(g, g % _NBUF)


def kernel(input_ids, table):
    ids = input_ids.reshape(-1).astype(jnp.int32)
    out = _emb_lookup(ids, table)
    return out.reshape(*input_ids.shape, _D)
